# baseline (speedup 1.0000x reference)
import jax
import jax.numpy as jnp
from jax.experimental import pallas as pl
from jax.experimental.pallas import tpu as pltpu

EPS = 1e-6
L = 128
BB = 2


def _dot(a, b, dims):
    return jax.lax.dot_general(a, b, (dims, ((), ())),
                               preferred_element_type=jnp.float32)


def _mm(a, b):
    return _dot(a, b, ((1,), (0,)))


def _mm_t(a, b):
    return _dot(a, b, ((1,), (1,)))


def _chunk_kernel(x_ref, w5, b5, wo, bo, out_ref, *Ss):
    j = pl.program_id(1)

    @pl.when(j == 0)
    def _():
        for S in Ss:
            S[:] = jnp.zeros_like(S)

    C = w5.shape[0]
    row = jax.lax.broadcasted_iota(jnp.int32, (L, L), 0)
    col = jax.lax.broadcasted_iota(jnp.int32, (L, L), 1)
    tril = (row >= col).astype(jnp.float32)
    eyeL = (row == col).astype(jnp.float32)

    def one(bi):
        xc = x_ref[bi]
        Z = _mm(xc, w5[:]) + b5[0]
        pre_q = Z[:, :C]
        pre_k = Z[:, C:2 * C]
        pre_v = Z[:, 2 * C:3 * C]
        ag = jnp.mean(jax.nn.sigmoid(Z[:, 3 * C:4 * C]), axis=-1,
                      keepdims=True)
        bg = jnp.mean(jax.nn.sigmoid(Z[:, 4 * C:]), axis=-1, keepdims=True)

        def silu(t):
            return t * jax.nn.sigmoid(t)

        def l2n(t):
            return t / (jnp.sqrt(jnp.sum(t * t, axis=-1, keepdims=True)) + EPS)

        qc = l2n(silu(pre_q))
        kc = l2n(silu(pre_k))
        vc = silu(pre_v)

        la = jnp.log(jnp.maximum(ag, 1e-30))
        Lc = _mm(tril, la)
        A = jnp.exp(Lc)
        D = Lc - Lc.reshape(1, L)
        G_strict = jnp.exp(jnp.where(row > col, D, -1e30))
        G_incl = G_strict + eyeL

        St = Ss[bi][:]
        KS0 = _mm_t(kc, St)
        RHS = bg * (vc - A * KS0)
        Nm = bg * (G_strict * _mm_t(kc, kc))
        P = eyeL - Nm
        Npow = Nm
        for _ in range(L.bit_length() - 2):
            Npow = _mm(Npow, Npow)
            P = _mm(P, eyeL + Npow)
        U = _mm(P, RHS)

        Pm = G_incl * _mm_t(qc, kc)
        O = A * _mm_t(qc, St) + _mm(Pm, U)
        out_ref[bi] = _mm(O, wo[:]) + bo[0]

        lcl = Lc[L - 1, 0]
        gam = jnp.exp(lcl - Lc)
        Ss[bi][:] = jnp.exp(lcl) * St + _dot(U * gam, kc, ((0,), (0,)))

    for bi in range(BB):
        one(bi)


@jax.jit
def kernel(x, Wq, bq, Wk, bk, Wv, bv, Wa, ba, Wb, bb, Wo, bo):
    B, N, C = x.shape
    grid = (B // BB, N // L)
    W5 = jnp.concatenate([Wq.T, Wk.T, Wv.T, Wa.T, Wb.T], axis=1)
    b5 = jnp.concatenate([bq, bk, bv, ba, bb]).reshape(1, 5 * C)
    xspec = pl.BlockSpec((BB, L, C), lambda b, j: (b, j, 0))
    out = pl.pallas_call(
        _chunk_kernel,
        grid=grid,
        in_specs=[xspec,
                  pl.BlockSpec((C, 5 * C), lambda b, j: (0, 0)),
                  pl.BlockSpec((1, 5 * C), lambda b, j: (0, 0)),
                  pl.BlockSpec((C, C), lambda b, j: (0, 0)),
                  pl.BlockSpec((1, C), lambda b, j: (0, 0))],
        out_specs=xspec,
        out_shape=jax.ShapeDtypeStruct((B, N, C), jnp.float32),
        scratch_shapes=[pltpu.VMEM((C, C), jnp.float32) for _ in range(BB)],
        compiler_params=pltpu.CompilerParams(
            dimension_semantics=("parallel", "arbitrary")),
    )(x, W5, b5, Wo.T, bo.reshape(1, C))
    return out


# speedup vs baseline: 1.3608x; 1.3608x over previous
import jax
import jax.numpy as jnp
from jax.experimental import pallas as pl
from jax.experimental.pallas import tpu as pltpu

EPS = 1e-6
L = 128
BB = 2


def _dot(a, b, dims):
    return jax.lax.dot_general(a, b, (dims, ((), ())),
                               preferred_element_type=jnp.float32)


def _mm(a, b):
    return _dot(a, b, ((1,), (0,)))


def _mm_t(a, b):
    return _dot(a, b, ((1,), (1,)))


def _chunk_kernel(x_ref, w5, b5, wo, bo, out_ref, *Ss):
    j = pl.program_id(1)

    @pl.when(j == 0)
    def _():
        for S in Ss:
            S[:] = jnp.zeros_like(S)

    C = w5.shape[0]
    row = jax.lax.broadcasted_iota(jnp.int32, (L, L), 0)
    col = jax.lax.broadcasted_iota(jnp.int32, (L, L), 1)
    tril = (row >= col).astype(jnp.float32)
    eyeL = (row == col).astype(jnp.float32)

    def silu(t):
        return t * jax.nn.sigmoid(t)

    def l2n(t):
        return t / (jnp.sqrt(jnp.sum(t * t, axis=-1, keepdims=True)) + EPS)

    def pre(bi):
        xc = x_ref[bi]
        Z = _mm(xc, w5[:]) + b5[0]
        qc = l2n(silu(Z[:, :C]))
        kc = l2n(silu(Z[:, C:2 * C]))
        vc = silu(Z[:, 2 * C:3 * C])
        ag = jnp.mean(jax.nn.sigmoid(Z[:, 3 * C:4 * C]), axis=-1,
                      keepdims=True)
        bg = jnp.mean(jax.nn.sigmoid(Z[:, 4 * C:]), axis=-1, keepdims=True)

        la = jnp.log(jnp.maximum(ag, 1e-30))
        Lc = _mm(tril, la)
        A = jnp.exp(Lc)
        D = Lc - Lc.reshape(1, L)
        G_strict = jnp.exp(jnp.where(row > col, D, -1e30))

        St = Ss[bi][:]
        KS0 = _mm_t(kc, St)
        RHS = bg * (vc - A * KS0)
        Nm = bg * (G_strict * _mm_t(kc, kc))
        Pm = (G_strict + eyeL) * _mm_t(qc, kc)
        QS0 = _mm_t(qc, St)
        return dict(qc=qc, kc=kc, A=A, Lc=Lc, St=St, RHS=RHS, Nm=Nm, Pm=Pm,
                    QS0=QS0)

    s = [pre(bi) for bi in range(BB)]

    P = [eyeL - s[bi]["Nm"] for bi in range(BB)]
    Npow = [s[bi]["Nm"] for bi in range(BB)]
    for _ in range(L.bit_length() - 2):
        Npow = [_mm(n, n) for n in Npow]
        P = [_mm(p, eyeL + n) for p, n in zip(P, Npow)]
    U = [_mm(P[bi], s[bi]["RHS"]) for bi in range(BB)]

    for bi in range(BB):
        d = s[bi]
        O = d["A"] * d["QS0"] + _mm(d["Pm"], U[bi])
        out_ref[bi] = _mm(O, wo[:]) + bo[0]
        lcl = d["Lc"][L - 1, 0]
        gam = jnp.exp(lcl - d["Lc"])
        Ss[bi][:] = jnp.exp(lcl) * d["St"] + _dot(U[bi] * gam, d["kc"],
                                                  ((0,), (0,)))


@jax.jit
def kernel(x, Wq, bq, Wk, bk, Wv, bv, Wa, ba, Wb, bb, Wo, bo):
    B, N, C = x.shape
    grid = (B // BB, N // L)
    W5 = jnp.concatenate([Wq.T, Wk.T, Wv.T, Wa.T, Wb.T], axis=1)
    b5 = jnp.concatenate([bq, bk, bv, ba, bb]).reshape(1, 5 * C)
    xspec = pl.BlockSpec((BB, L, C), lambda b, j: (b, j, 0))
    out = pl.pallas_call(
        _chunk_kernel,
        grid=grid,
        in_specs=[xspec,
                  pl.BlockSpec((C, 5 * C), lambda b, j: (0, 0)),
                  pl.BlockSpec((1, 5 * C), lambda b, j: (0, 0)),
                  pl.BlockSpec((C, C), lambda b, j: (0, 0)),
                  pl.BlockSpec((1, C), lambda b, j: (0, 0))],
        out_specs=xspec,
        out_shape=jax.ShapeDtypeStruct((B, N, C), jnp.float32),
        scratch_shapes=[pltpu.VMEM((C, C), jnp.float32) for _ in range(BB)],
        compiler_params=pltpu.CompilerParams(
            dimension_semantics=("parallel", "arbitrary")),
    )(x, W5, b5, Wo.T, bo.reshape(1, C))
    return out


# speedup vs baseline: 1.7986x; 1.3217x over previous
import jax
import jax.numpy as jnp
from jax.experimental import pallas as pl
from jax.experimental.pallas import tpu as pltpu

EPS = 1e-6
L = 128
BB = 4


def _dot(a, b, dims):
    return jax.lax.dot_general(a, b, (dims, ((), ())),
                               preferred_element_type=jnp.float32)


def _mm(a, b):
    return _dot(a, b, ((1,), (0,)))


def _mm_t(a, b):
    return _dot(a, b, ((1,), (1,)))


def _chunk_kernel(x_ref, w5, b5, wo, bo, out_ref, *Ss):
    j = pl.program_id(1)

    @pl.when(j == 0)
    def _():
        for S in Ss:
            S[:] = jnp.zeros_like(S)

    C = w5.shape[0]
    row = jax.lax.broadcasted_iota(jnp.int32, (L, L), 0)
    col = jax.lax.broadcasted_iota(jnp.int32, (L, L), 1)
    tril = (row >= col).astype(jnp.float32)
    eyeL = (row == col).astype(jnp.float32)

    def silu(t):
        return t * jax.nn.sigmoid(t)

    def l2n(t):
        return t / (jnp.sqrt(jnp.sum(t * t, axis=-1, keepdims=True)) + EPS)

    def pre(bi):
        xc = x_ref[bi]
        Z = _mm(xc, w5[:]) + b5[0]
        qc = l2n(silu(Z[:, :C]))
        kc = l2n(silu(Z[:, C:2 * C]))
        vc = silu(Z[:, 2 * C:3 * C])
        ag = jnp.mean(jax.nn.sigmoid(Z[:, 3 * C:4 * C]), axis=-1,
                      keepdims=True)
        bg = jnp.mean(jax.nn.sigmoid(Z[:, 4 * C:]), axis=-1, keepdims=True)

        la = jnp.log(jnp.maximum(ag, 1e-30))
        Lc = _mm(tril, la)
        A = jnp.exp(Lc)
        D = Lc - Lc.reshape(1, L)
        G_strict = jnp.exp(jnp.where(row > col, D, -1e30))

        St = Ss[bi][:]
        KS0 = _mm_t(kc, St)
        RHS = bg * (vc - A * KS0)
        Nm = bg * (G_strict * _mm_t(kc, kc))
        Pm = (G_strict + eyeL) * _mm_t(qc, kc)
        QS0 = _mm_t(qc, St)
        return dict(qc=qc, kc=kc, A=A, Lc=Lc, St=St, RHS=RHS, Nm=Nm, Pm=Pm,
                    QS0=QS0)

    s = [pre(bi) for bi in range(BB)]

    P = [eyeL - s[bi]["Nm"] for bi in range(BB)]
    Npow = [s[bi]["Nm"] for bi in range(BB)]
    for _ in range(L.bit_length() - 2):
        Npow = [_mm(n, n) for n in Npow]
        P = [_mm(p, eyeL + n) for p, n in zip(P, Npow)]
    U = [_mm(P[bi], s[bi]["RHS"]) for bi in range(BB)]

    for bi in range(BB):
        d = s[bi]
        O = d["A"] * d["QS0"] + _mm(d["Pm"], U[bi])
        out_ref[bi] = _mm(O, wo[:]) + bo[0]
        lcl = d["Lc"][L - 1, 0]
        gam = jnp.exp(lcl - d["Lc"])
        Ss[bi][:] = jnp.exp(lcl) * d["St"] + _dot(U[bi] * gam, d["kc"],
                                                  ((0,), (0,)))


@jax.jit
def kernel(x, Wq, bq, Wk, bk, Wv, bv, Wa, ba, Wb, bb, Wo, bo):
    B, N, C = x.shape
    grid = (B // BB, N // L)
    W5 = jnp.concatenate([Wq.T, Wk.T, Wv.T, Wa.T, Wb.T], axis=1)
    b5 = jnp.concatenate([bq, bk, bv, ba, bb]).reshape(1, 5 * C)
    xspec = pl.BlockSpec((BB, L, C), lambda b, j: (b, j, 0))
    out = pl.pallas_call(
        _chunk_kernel,
        grid=grid,
        in_specs=[xspec,
                  pl.BlockSpec((C, 5 * C), lambda b, j: (0, 0)),
                  pl.BlockSpec((1, 5 * C), lambda b, j: (0, 0)),
                  pl.BlockSpec((C, C), lambda b, j: (0, 0)),
                  pl.BlockSpec((1, C), lambda b, j: (0, 0))],
        out_specs=xspec,
        out_shape=jax.ShapeDtypeStruct((B, N, C), jnp.float32),
        scratch_shapes=[pltpu.VMEM((C, C), jnp.float32) for _ in range(BB)],
        compiler_params=pltpu.CompilerParams(
            dimension_semantics=("arbitrary", "arbitrary")),
    )(x, W5, b5, Wo.T, bo.reshape(1, C))
    return out


# speedup vs baseline: 2.0242x; 1.1255x over previous
import jax
import jax.numpy as jnp
from jax.experimental import pallas as pl
from jax.experimental.pallas import tpu as pltpu

EPS = 1e-6
L = 128
BB = 8


def _dot(a, b, dims):
    return jax.lax.dot_general(a, b, (dims, ((), ())),
                               preferred_element_type=jnp.float32)


def _mm(a, b):
    return _dot(a, b, ((1,), (0,)))


def _mm_t(a, b):
    return _dot(a, b, ((1,), (1,)))


def _chunk_kernel(x_ref, w5, b5, wo, bo, out_ref, *Ss):
    j = pl.program_id(1)

    @pl.when(j == 0)
    def _():
        for S in Ss:
            S[:] = jnp.zeros_like(S)

    C = w5.shape[0]
    row = jax.lax.broadcasted_iota(jnp.int32, (L, L), 0)
    col = jax.lax.broadcasted_iota(jnp.int32, (L, L), 1)
    tril = (row >= col).astype(jnp.float32)
    eyeL = (row == col).astype(jnp.float32)

    def silu(t):
        return t * jax.nn.sigmoid(t)

    def l2n(t):
        return t / (jnp.sqrt(jnp.sum(t * t, axis=-1, keepdims=True)) + EPS)

    def pre(bi):
        xc = x_ref[bi]
        Z = _mm(xc, w5[:]) + b5[0]
        qc = l2n(silu(Z[:, :C]))
        kc = l2n(silu(Z[:, C:2 * C]))
        vc = silu(Z[:, 2 * C:3 * C])
        ag = jnp.mean(jax.nn.sigmoid(Z[:, 3 * C:4 * C]), axis=-1,
                      keepdims=True)
        bg = jnp.mean(jax.nn.sigmoid(Z[:, 4 * C:]), axis=-1, keepdims=True)

        la = jnp.log(jnp.maximum(ag, 1e-30))
        Lc = _mm(tril, la)
        A = jnp.exp(Lc)
        D = Lc - Lc.reshape(1, L)
        G_strict = jnp.exp(jnp.where(row > col, D, -1e30))

        St = Ss[bi][:]
        KS0 = _mm_t(kc, St)
        RHS = bg * (vc - A * KS0)
        Nm = bg * (G_strict * _mm_t(kc, kc))
        Pm = (G_strict + eyeL) * _mm_t(qc, kc)
        QS0 = _mm_t(qc, St)
        return dict(qc=qc, kc=kc, A=A, Lc=Lc, St=St, RHS=RHS, Nm=Nm, Pm=Pm,
                    QS0=QS0)

    s = [pre(bi) for bi in range(BB)]

    P = [eyeL - s[bi]["Nm"] for bi in range(BB)]
    Npow = [s[bi]["Nm"] for bi in range(BB)]
    for _ in range(L.bit_length() - 2):
        Npow = [_mm(n, n) for n in Npow]
        P = [_mm(p, eyeL + n) for p, n in zip(P, Npow)]
    U = [_mm(P[bi], s[bi]["RHS"]) for bi in range(BB)]

    for bi in range(BB):
        d = s[bi]
        O = d["A"] * d["QS0"] + _mm(d["Pm"], U[bi])
        out_ref[bi] = _mm(O, wo[:]) + bo[0]
        lcl = d["Lc"][L - 1, 0]
        gam = jnp.exp(lcl - d["Lc"])
        Ss[bi][:] = jnp.exp(lcl) * d["St"] + _dot(U[bi] * gam, d["kc"],
                                                  ((0,), (0,)))


@jax.jit
def kernel(x, Wq, bq, Wk, bk, Wv, bv, Wa, ba, Wb, bb, Wo, bo):
    B, N, C = x.shape
    grid = (B // BB, N // L)
    W5 = jnp.concatenate([Wq.T, Wk.T, Wv.T, Wa.T, Wb.T], axis=1)
    b5 = jnp.concatenate([bq, bk, bv, ba, bb]).reshape(1, 5 * C)
    xspec = pl.BlockSpec((BB, L, C), lambda b, j: (b, j, 0))
    out = pl.pallas_call(
        _chunk_kernel,
        grid=grid,
        in_specs=[xspec,
                  pl.BlockSpec((C, 5 * C), lambda b, j: (0, 0)),
                  pl.BlockSpec((1, 5 * C), lambda b, j: (0, 0)),
                  pl.BlockSpec((C, C), lambda b, j: (0, 0)),
                  pl.BlockSpec((1, C), lambda b, j: (0, 0))],
        out_specs=xspec,
        out_shape=jax.ShapeDtypeStruct((B, N, C), jnp.float32),
        scratch_shapes=[pltpu.VMEM((C, C), jnp.float32) for _ in range(BB)],
        compiler_params=pltpu.CompilerParams(
            dimension_semantics=("arbitrary", "arbitrary")),
    )(x, W5, b5, Wo.T, bo.reshape(1, C))
    return out


# speedup vs baseline: 2.0297x; 1.0027x over previous
import jax
import jax.numpy as jnp
from jax.experimental import pallas as pl
from jax.experimental.pallas import tpu as pltpu

EPS = 1e-6
L = 128
BB = 8


def _dot(a, b, dims, bf=False):
    if bf:
        a = a.astype(jnp.bfloat16)
        b = b.astype(jnp.bfloat16)
    return jax.lax.dot_general(a, b, (dims, ((), ())),
                               preferred_element_type=jnp.float32)


def _mm(a, b, bf=False):
    return _dot(a, b, ((1,), (0,)), bf)


def _mm_t(a, b, bf=False):
    return _dot(a, b, ((1,), (1,)), bf)


def _chunk_kernel(x_ref, w5, b5, wo, bo, out_ref, *Ss):
    j = pl.program_id(1)

    @pl.when(j == 0)
    def _():
        for S in Ss:
            S[:] = jnp.zeros_like(S)

    C = w5.shape[0]
    row = jax.lax.broadcasted_iota(jnp.int32, (L, L), 0)
    col = jax.lax.broadcasted_iota(jnp.int32, (L, L), 1)
    tril = (row >= col).astype(jnp.float32)
    eyeL = (row == col).astype(jnp.float32)

    def silu(t):
        return t * jax.nn.sigmoid(t)

    def l2n(t):
        return t / (jnp.sqrt(jnp.sum(t * t, axis=-1, keepdims=True)) + EPS)

    def pre(bi):
        xc = x_ref[bi]
        Z = _mm(xc, w5[:], bf=True) + b5[0]
        qc = l2n(silu(Z[:, :C]))
        kc = l2n(silu(Z[:, C:2 * C]))
        vc = silu(Z[:, 2 * C:3 * C])
        ag = jnp.mean(jax.nn.sigmoid(Z[:, 3 * C:4 * C]), axis=-1,
                      keepdims=True)
        bg = jnp.mean(jax.nn.sigmoid(Z[:, 4 * C:]), axis=-1, keepdims=True)

        la = jnp.log(jnp.maximum(ag, 1e-30))
        Lc = _mm(tril, la)
        A = jnp.exp(Lc)
        D = Lc - Lc.reshape(1, L)
        G_strict = jnp.exp(jnp.where(row > col, D, -1e30))

        St = Ss[bi][:]
        KS0 = _mm_t(kc, St, bf=True)
        RHS = bg * (vc - A * KS0)
        Nm = bg * (G_strict * _mm_t(kc, kc, bf=True))
        Pm = (G_strict + eyeL) * _mm_t(qc, kc, bf=True)
        QS0 = _mm_t(qc, St, bf=True)
        return dict(qc=qc, kc=kc, A=A, Lc=Lc, St=St, RHS=RHS, Nm=Nm, Pm=Pm,
                    QS0=QS0)

    s = [pre(bi) for bi in range(BB)]

    P = [eyeL - s[bi]["Nm"] for bi in range(BB)]
    Npow = [s[bi]["Nm"] for bi in range(BB)]
    for _ in range(L.bit_length() - 2):
        Npow = [_mm(n, n) for n in Npow]
        P = [_mm(p, eyeL + n) for p, n in zip(P, Npow)]
    U = [_mm(P[bi], s[bi]["RHS"]) for bi in range(BB)]

    for bi in range(BB):
        d = s[bi]
        O = d["A"] * d["QS0"] + _mm(d["Pm"], U[bi], bf=True)
        out_ref[bi] = _mm(O, wo[:], bf=True) + bo[0]
        lcl = d["Lc"][L - 1, 0]
        gam = jnp.exp(lcl - d["Lc"])
        Ss[bi][:] = jnp.exp(lcl) * d["St"] + _dot(U[bi] * gam, d["kc"],
                                                  ((0,), (0,)), bf=True)


@jax.jit
def kernel(x, Wq, bq, Wk, bk, Wv, bv, Wa, ba, Wb, bb, Wo, bo):
    B, N, C = x.shape
    grid = (B // BB, N // L)
    W5 = jnp.concatenate([Wq.T, Wk.T, Wv.T, Wa.T, Wb.T], axis=1)
    b5 = jnp.concatenate([bq, bk, bv, ba, bb]).reshape(1, 5 * C)
    xspec = pl.BlockSpec((BB, L, C), lambda b, j: (b, j, 0))
    out = pl.pallas_call(
        _chunk_kernel,
        grid=grid,
        in_specs=[xspec,
                  pl.BlockSpec((C, 5 * C), lambda b, j: (0, 0)),
                  pl.BlockSpec((1, 5 * C), lambda b, j: (0, 0)),
                  pl.BlockSpec((C, C), lambda b, j: (0, 0)),
                  pl.BlockSpec((1, C), lambda b, j: (0, 0))],
        out_specs=xspec,
        out_shape=jax.ShapeDtypeStruct((B, N, C), jnp.float32),
        scratch_shapes=[pltpu.VMEM((C, C), jnp.float32) for _ in range(BB)],
        compiler_params=pltpu.CompilerParams(
            dimension_semantics=("arbitrary", "arbitrary")),
    )(x, W5, b5, Wo.T, bo.reshape(1, C))
    return out


# speedup vs baseline: 2.0739x; 1.0218x over previous
import jax
import jax.numpy as jnp
from jax.experimental import pallas as pl
from jax.experimental.pallas import tpu as pltpu

EPS = 1e-6
L = 128
BB = 8
CH = 2


def _dot(a, b, dims, bf=False):
    if bf:
        a = a.astype(jnp.bfloat16)
        b = b.astype(jnp.bfloat16)
    return jax.lax.dot_general(a, b, (dims, ((), ())),
                               preferred_element_type=jnp.float32)


def _mm(a, b, bf=False):
    return _dot(a, b, ((1,), (0,)), bf)


def _mm_t(a, b, bf=False):
    return _dot(a, b, ((1,), (1,)), bf)


def _chunk_kernel(x_ref, w5, b5, wo, bo, out_ref, *Ss):
    j = pl.program_id(0)

    C = w5.shape[0]
    row = jax.lax.broadcasted_iota(jnp.int32, (L, L), 0)
    col = jax.lax.broadcasted_iota(jnp.int32, (L, L), 1)
    tril = (row >= col).astype(jnp.float32)
    eyeL = (row == col).astype(jnp.float32)

    def silu(t):
        return t * jax.nn.sigmoid(t)

    def l2n(t):
        return t / (jnp.sqrt(jnp.sum(t * t, axis=-1, keepdims=True)) + EPS)

    zero = jnp.zeros((C, C), jnp.float32)
    St = [jnp.where(j == 0, zero, Ss[bi][:]) for bi in range(BB)]

    for ch in range(CH):
        def pre(bi):
            xc = x_ref[bi, ch * L:(ch + 1) * L]
            Z = _mm(xc, w5[:], bf=True) + b5[0]
            qc = l2n(silu(Z[:, :C]))
            kc = l2n(silu(Z[:, C:2 * C]))
            vc = silu(Z[:, 2 * C:3 * C])
            ag = jnp.mean(jax.nn.sigmoid(Z[:, 3 * C:4 * C]), axis=-1,
                          keepdims=True)
            bg = jnp.mean(jax.nn.sigmoid(Z[:, 4 * C:]), axis=-1,
                          keepdims=True)

            la = jnp.log(jnp.maximum(ag, 1e-30))
            Lc = _mm(tril, la)
            A = jnp.exp(Lc)
            D = Lc - Lc.reshape(1, L)
            G_strict = jnp.exp(jnp.where(row > col, D, -1e30))

            KS0 = _mm_t(kc, St[bi], bf=True)
            RHS = bg * (vc - A * KS0)
            Nm = bg * (G_strict * _mm_t(kc, kc, bf=True))
            Pm = (G_strict + eyeL) * _mm_t(qc, kc, bf=True)
            QS0 = _mm_t(qc, St[bi], bf=True)
            return dict(kc=kc, A=A, Lc=Lc, RHS=RHS, Nm=Nm, Pm=Pm, QS0=QS0)

        s = [pre(bi) for bi in range(BB)]

        P = [eyeL - s[bi]["Nm"] for bi in range(BB)]
        Npow = [s[bi]["Nm"] for bi in range(BB)]
        for _ in range(L.bit_length() - 2):
            Npow = [_mm(n, n) for n in Npow]
            P = [_mm(p, eyeL + n) for p, n in zip(P, Npow)]
        U = [_mm(P[bi], s[bi]["RHS"]) for bi in range(BB)]

        for bi in range(BB):
            d = s[bi]
            O = d["A"] * d["QS0"] + _mm(d["Pm"], U[bi], bf=True)
            out_ref[bi, ch * L:(ch + 1) * L] = _mm(O, wo[:], bf=True) + bo[0]
            lcl = d["Lc"][L - 1, 0]
            gam = jnp.exp(lcl - d["Lc"])
            St[bi] = jnp.exp(lcl) * St[bi] + _dot(U[bi] * gam, d["kc"],
                                                  ((0,), (0,)), bf=True)

    for bi in range(BB):
        Ss[bi][:] = St[bi]


@jax.jit
def kernel(x, Wq, bq, Wk, bk, Wv, bv, Wa, ba, Wb, bb, Wo, bo):
    B, N, C = x.shape
    grid = (N // (CH * L),)
    W5 = jnp.concatenate([Wq.T, Wk.T, Wv.T, Wa.T, Wb.T], axis=1)
    b5 = jnp.concatenate([bq, bk, bv, ba, bb]).reshape(1, 5 * C)
    xspec = pl.BlockSpec((BB, CH * L, C), lambda j: (0, j, 0))
    out = pl.pallas_call(
        _chunk_kernel,
        grid=grid,
        in_specs=[xspec,
                  pl.BlockSpec((C, 5 * C), lambda j: (0, 0)),
                  pl.BlockSpec((1, 5 * C), lambda j: (0, 0)),
                  pl.BlockSpec((C, C), lambda j: (0, 0)),
                  pl.BlockSpec((1, C), lambda j: (0, 0))],
        out_specs=xspec,
        out_shape=jax.ShapeDtypeStruct((B, N, C), jnp.float32),
        scratch_shapes=[pltpu.VMEM((C, C), jnp.float32) for _ in range(BB)],
        compiler_params=pltpu.CompilerParams(
            dimension_semantics=("arbitrary",)),
    )(x, W5, b5, Wo.T, bo.reshape(1, C))
    return out


# speedup vs baseline: 2.1100x; 1.0174x over previous
import jax
import jax.numpy as jnp
from jax.experimental import pallas as pl
from jax.experimental.pallas import tpu as pltpu

EPS = 1e-6
L = 128
BB = 8
CH = 2


def _dot(a, b, dims):
    return jax.lax.dot_general(a, b, (dims, ((), ())),
                               preferred_element_type=jnp.float32)


def _mm(a, b):
    return _dot(a, b, ((1,), (0,)))


def _mm_t(a, b):
    return _dot(a, b, ((1,), (1,)))


def _chunk_kernel(x_ref, w5, b5, wo, bo, out_ref, *Ss):
    j = pl.program_id(1)

    @pl.when(j == 0)
    def _():
        for S in Ss:
            S[:] = jnp.zeros_like(S)

    C = w5.shape[0]
    row = jax.lax.broadcasted_iota(jnp.int32, (L, L), 0)
    col = jax.lax.broadcasted_iota(jnp.int32, (L, L), 1)
    tril = (row >= col).astype(jnp.float32)
    eyeL = (row == col).astype(jnp.float32)
    H = L // 2
    eyeH = eyeL[:H, :H]
    w5b = w5[:]

    def silu(t):
        return t * jax.nn.sigmoid(t)

    def l2n(t):
        return t / (jnp.sqrt(jnp.sum(t * t, axis=-1, keepdims=True)) + EPS)

    for ch in range(CH):
        def pre(bi):
            xc = x_ref[bi, ch * L:(ch + 1) * L]
            Z = _mm(xc.astype(jnp.bfloat16), w5b) + b5[0]
            qb = l2n(silu(Z[:, :C])).astype(jnp.bfloat16)
            kb = l2n(silu(Z[:, C:2 * C])).astype(jnp.bfloat16)
            vc = silu(Z[:, 2 * C:3 * C])
            ag = jnp.mean(jax.nn.sigmoid(Z[:, 3 * C:4 * C]), axis=-1,
                          keepdims=True)
            bg = jnp.mean(jax.nn.sigmoid(Z[:, 4 * C:]), axis=-1,
                          keepdims=True)

            la = jnp.log(jnp.maximum(ag, 1e-30))
            Lc = _mm(tril, la)
            A = jnp.exp(Lc)
            D = Lc - Lc.reshape(1, L)
            G_strict = jnp.exp(jnp.where(row > col, D, -1e30))

            Sb = Ss[bi][:].astype(jnp.bfloat16)
            KS0 = _mm_t(kb, Sb)
            RHS = bg * (vc - A * KS0)
            Nm = bg * (G_strict * _mm_t(kb, kb))
            return dict(qb=qb, kb=kb, A=A, Lc=Lc, RHS=RHS, Nm=Nm,
                        G=G_strict, Sb=Sb)

        s = [pre(bi) for bi in range(BB)]

        Pb = [[eyeH - s[bi]["Nm"][d * H:(d + 1) * H, d * H:(d + 1) * H]
               for d in range(2)] for bi in range(BB)]
        Npow = [[-(Pb[bi][d] - eyeH) for d in range(2)] for bi in range(BB)]
        for _ in range(H.bit_length() - 2):
            Npow = [[_mm(n, n) for n in bn] for bn in Npow]
            Pb = [[_mm(p, eyeH + n) for p, n in zip(bp, bn)]
                  for bp, bn in zip(Pb, Npow)]
        U1 = [_mm(Pb[bi][0], s[bi]["RHS"][:H]) for bi in range(BB)]
        U2 = [_mm(Pb[bi][1],
                  s[bi]["RHS"][H:] - _mm(s[bi]["Nm"][H:, :H], U1[bi]))
              for bi in range(BB)]
        U = [jnp.concatenate([U1[bi], U2[bi]], axis=0) for bi in range(BB)]

        for bi in range(BB):
            d = s[bi]
            Pmb = ((d["G"] + eyeL) * _mm_t(d["qb"], d["kb"])) \
                .astype(jnp.bfloat16)
            O = d["A"] * _mm_t(d["qb"], d["Sb"]) \
                + _mm(Pmb, U[bi].astype(jnp.bfloat16))
            out_ref[bi, ch * L:(ch + 1) * L] = \
                _mm(O.astype(jnp.bfloat16), wo[:]) + bo[0]
            lcl = d["Lc"][L - 1, 0]
            gam = jnp.exp(lcl - d["Lc"])
            Ss[bi][:] = jnp.exp(lcl) * Ss[bi][:] + \
                _dot((U[bi] * gam).astype(jnp.bfloat16), d["kb"],
                     ((0,), (0,)))

    return


@jax.jit
def kernel(x, Wq, bq, Wk, bk, Wv, bv, Wa, ba, Wb, bb, Wo, bo):
    B, N, C = x.shape
    grid = (B // BB, N // (CH * L))
    W5 = jnp.concatenate([Wq.T, Wk.T, Wv.T, Wa.T, Wb.T],
                         axis=1).astype(jnp.bfloat16)
    b5 = jnp.concatenate([bq, bk, bv, ba, bb]).reshape(1, 5 * C)
    xspec = pl.BlockSpec((BB, CH * L, C), lambda b, j: (b, j, 0))
    out = pl.pallas_call(
        _chunk_kernel,
        grid=grid,
        in_specs=[xspec,
                  pl.BlockSpec((C, 5 * C), lambda b, j: (0, 0)),
                  pl.BlockSpec((1, 5 * C), lambda b, j: (0, 0)),
                  pl.BlockSpec((C, C), lambda b, j: (0, 0)),
                  pl.BlockSpec((1, C), lambda b, j: (0, 0))],
        out_specs=xspec,
        out_shape=jax.ShapeDtypeStruct((B, N, C), jnp.float32),
        scratch_shapes=[pltpu.VMEM((C, C), jnp.float32) for _ in range(BB)],
        compiler_params=pltpu.CompilerParams(
            dimension_semantics=("arbitrary", "arbitrary")),
    )(x, W5, b5, Wo.T.astype(jnp.bfloat16), bo.reshape(1, C))
    return out


# speedup vs baseline: 2.2878x; 1.0842x over previous
import jax
import jax.numpy as jnp
from jax.experimental import pallas as pl
from jax.experimental.pallas import tpu as pltpu

EPS = 1e-6
L = 128
BB = 8


def _dot(a, b, dims):
    return jax.lax.dot_general(a, b, (dims, ((), ())),
                               preferred_element_type=jnp.float32)


def _mm(a, b):
    return _dot(a, b, ((1,), (0,)))


def _mm_t(a, b):
    return _dot(a, b, ((1,), (1,)))


def _chunk_kernel(x_ref, w5, b5, wo, bo, out_ref, *Ss):
    j = pl.program_id(0)

    @pl.when(j == 0)
    def _():
        for S in Ss:
            S[:] = jnp.zeros_like(S)

    C = w5.shape[0]
    row = jax.lax.broadcasted_iota(jnp.int32, (L, L), 0)
    col = jax.lax.broadcasted_iota(jnp.int32, (L, L), 1)
    tril = (row >= col).astype(jnp.float32)
    eyeL = (row == col).astype(jnp.float32)
    H = L // 2
    eyeH = eyeL[:H, :H]

    def silu(t):
        return t * jax.nn.sigmoid(t)

    def l2n(t):
        return t / (jnp.sqrt(jnp.sum(t * t, axis=-1, keepdims=True)) + EPS)

    xb = x_ref[:].reshape(BB * L, C).astype(jnp.bfloat16)
    Z = _mm(xb, w5[:]) + b5[0]
    qb_all = l2n(silu(Z[:, :C])).astype(jnp.bfloat16)
    kb_all = l2n(silu(Z[:, C:2 * C])).astype(jnp.bfloat16)
    vc_all = silu(Z[:, 2 * C:3 * C])
    ag_all = jnp.mean(jax.nn.sigmoid(Z[:, 3 * C:4 * C]), axis=-1,
                      keepdims=True)
    bg_all = jnp.mean(jax.nn.sigmoid(Z[:, 4 * C:]), axis=-1, keepdims=True)
    la_all = jnp.log(jnp.maximum(ag_all, 1e-30))

    def pre(bi):
        sl = slice(bi * L, (bi + 1) * L)
        kb = kb_all[sl]
        bg = bg_all[sl]
        Lc = _mm(tril, la_all[sl])
        A = jnp.exp(Lc)
        D = Lc - Lc.reshape(1, L)
        G = jnp.exp(jnp.where(row > col, D, -1e30))
        Sb = Ss[bi][:].astype(jnp.bfloat16)
        KS0 = _mm_t(kb, Sb)
        RHS = bg * (vc_all[sl] - A * KS0)
        Nm = bg * (G * _mm_t(kb, kb))
        return dict(kb=kb, A=A, Lc=Lc, RHS=RHS, Nm=Nm, G=G, Sb=Sb)

    s = [pre(bi) for bi in range(BB)]

    Pb = [[eyeH - s[bi]["Nm"][d * H:(d + 1) * H, d * H:(d + 1) * H]
           for d in range(2)] for bi in range(BB)]
    Npow = [[-(Pb[bi][d] - eyeH) for d in range(2)] for bi in range(BB)]
    for _ in range(H.bit_length() - 2):
        Npow = [[_mm(n, n) for n in bn] for bn in Npow]
        Pb = [[_mm(p, eyeH + n) for p, n in zip(bp, bn)]
              for bp, bn in zip(Pb, Npow)]
    U1 = [_mm(Pb[bi][0], s[bi]["RHS"][:H]) for bi in range(BB)]
    U2 = [_mm(Pb[bi][1],
              s[bi]["RHS"][H:] - _mm(s[bi]["Nm"][H:, :H], U1[bi]))
          for bi in range(BB)]
    U = [jnp.concatenate([U1[bi], U2[bi]], axis=0) for bi in range(BB)]

    for bi in range(BB):
        d = s[bi]
        sl = slice(bi * L, (bi + 1) * L)
        qb = qb_all[sl]
        Pmb = ((d["G"] + eyeL) * _mm_t(qb, d["kb"])).astype(jnp.bfloat16)
        O = d["A"] * _mm_t(qb, d["Sb"]) \
            + _mm(Pmb, U[bi].astype(jnp.bfloat16))
        out_ref[bi] = _mm(O.astype(jnp.bfloat16), wo[:]) + bo[0]
        lcl = d["Lc"][L - 1, 0]
        gam = jnp.exp(lcl - d["Lc"])
        Ss[bi][:] = jnp.exp(lcl) * Ss[bi][:] + \
            _dot((U[bi] * gam).astype(jnp.bfloat16), d["kb"], ((0,), (0,)))


@jax.jit
def kernel(x, Wq, bq, Wk, bk, Wv, bv, Wa, ba, Wb, bb, Wo, bo):
    B, N, C = x.shape
    grid = (N // L,)
    W5 = jnp.concatenate([Wq.T, Wk.T, Wv.T, Wa.T, Wb.T],
                         axis=1).astype(jnp.bfloat16)
    b5 = jnp.concatenate([bq, bk, bv, ba, bb]).reshape(1, 5 * C)
    xspec = pl.BlockSpec((BB, L, C), lambda j: (0, j, 0))
    out = pl.pallas_call(
        _chunk_kernel,
        grid=grid,
        in_specs=[xspec,
                  pl.BlockSpec((C, 5 * C), lambda j: (0, 0)),
                  pl.BlockSpec((1, 5 * C), lambda j: (0, 0)),
                  pl.BlockSpec((C, C), lambda j: (0, 0)),
                  pl.BlockSpec((1, C), lambda j: (0, 0))],
        out_specs=xspec,
        out_shape=jax.ShapeDtypeStruct((B, N, C), jnp.float32),
        scratch_shapes=[pltpu.VMEM((C, C), jnp.float32) for _ in range(BB)],
        compiler_params=pltpu.CompilerParams(
            dimension_semantics=("arbitrary",)),
    )(x, W5, b5, Wo.T.astype(jnp.bfloat16), bo.reshape(1, C))
    return out


# speedup vs baseline: 2.5427x; 1.1114x over previous
import jax
import jax.numpy as jnp
from jax.experimental import pallas as pl
from jax.experimental.pallas import tpu as pltpu

EPS = 1e-6
L = 128
BB = 8


def _dot(a, b, dims):
    return jax.lax.dot_general(a, b, (dims, ((), ())),
                               preferred_element_type=jnp.float32)


def _mm(a, b):
    return _dot(a, b, ((1,), (0,)))


def _mm_t(a, b):
    return _dot(a, b, ((1,), (1,)))


def _chunk_kernel(x_ref, w5, b5, wo, bo, out_ref, *Ss):
    j = pl.program_id(0)

    @pl.when(j == 0)
    def _():
        for S in Ss:
            S[:] = jnp.zeros_like(S)

    C = w5.shape[0]
    row = jax.lax.broadcasted_iota(jnp.int32, (L, L), 0)
    col = jax.lax.broadcasted_iota(jnp.int32, (L, L), 1)
    tril = (row >= col).astype(jnp.float32)
    eyeL = (row == col).astype(jnp.float32)
    H = L // 2
    eyeH = eyeL[:H, :H]

    def silu(t):
        return t * jax.nn.sigmoid(t)

    def l2n(t):
        return t / (jnp.sqrt(jnp.sum(t * t, axis=-1, keepdims=True)) + EPS)

    xb = x_ref[:].reshape(BB * L, C).astype(jnp.bfloat16)
    Z = _mm(xb, w5[:]) + b5[0]
    qb_all = l2n(silu(Z[:, :C])).astype(jnp.bfloat16)
    kb_all = l2n(silu(Z[:, C:2 * C])).astype(jnp.bfloat16)
    vc_all = silu(Z[:, 2 * C:3 * C])
    ag_all = jnp.mean(jax.nn.sigmoid(Z[:, 3 * C:4 * C]), axis=-1,
                      keepdims=True)
    bg_all = jnp.mean(jax.nn.sigmoid(Z[:, 4 * C:]), axis=-1, keepdims=True)
    la_all = jnp.log(jnp.maximum(ag_all, 1e-30))

    def pre(bi):
        sl = slice(bi * L, (bi + 1) * L)
        kb = kb_all[sl]
        bg = bg_all[sl]
        Lc = _mm(tril, la_all[sl])
        A = jnp.exp(Lc)
        D = Lc - Lc.reshape(1, L)
        G = jnp.exp(jnp.where(row > col, D, -1e30))
        Sb = Ss[bi][:].astype(jnp.bfloat16)
        KS0 = _mm_t(kb, Sb)
        RHS = bg * (vc_all[sl] - A * KS0)
        Nm = bg * (G * _mm_t(kb, kb))
        return dict(kb=kb, A=A, Lc=Lc, RHS=RHS, Nm=Nm, G=G, Sb=Sb)

    s = [pre(bi) for bi in range(BB)]

    Pb = [[eyeH - s[bi]["Nm"][d * H:(d + 1) * H, d * H:(d + 1) * H]
           for d in range(2)] for bi in range(BB)]
    Npow = [[-(Pb[bi][d] - eyeH) for d in range(2)] for bi in range(BB)]
    for _ in range(H.bit_length() - 2):
        Npow = [[_mm(n, n) for n in bn] for bn in Npow]
        Pb = [[_mm(p, eyeH + n) for p, n in zip(bp, bn)]
              for bp, bn in zip(Pb, Npow)]
    U1 = [_mm(Pb[bi][0], s[bi]["RHS"][:H]) for bi in range(BB)]
    U2 = [_mm(Pb[bi][1],
              s[bi]["RHS"][H:] - _mm(s[bi]["Nm"][H:, :H], U1[bi]))
          for bi in range(BB)]
    U = [jnp.concatenate([U1[bi], U2[bi]], axis=0) for bi in range(BB)]

    Os = []
    for bi in range(BB):
        d = s[bi]
        sl = slice(bi * L, (bi + 1) * L)
        qb = qb_all[sl]
        Pmb = ((d["G"] + eyeL) * _mm_t(qb, d["kb"])).astype(jnp.bfloat16)
        O = d["A"] * _mm_t(qb, d["Sb"]) \
            + _mm(Pmb, U[bi].astype(jnp.bfloat16))
        Os.append(O.astype(jnp.bfloat16))
        lcl = d["Lc"][L - 1, 0]
        gam = jnp.exp(lcl - d["Lc"])
        Ss[bi][:] = jnp.exp(lcl) * Ss[bi][:] + \
            _dot((U[bi] * gam).astype(jnp.bfloat16), d["kb"], ((0,), (0,)))

    O_all = jnp.concatenate(Os, axis=0)
    out_ref[:] = (_mm(O_all, wo[:]) + bo[0]).reshape(BB, L, C)


@jax.jit
def kernel(x, Wq, bq, Wk, bk, Wv, bv, Wa, ba, Wb, bb, Wo, bo):
    B, N, C = x.shape
    grid = (N // L,)
    W5 = jnp.concatenate([Wq.T, Wk.T, Wv.T, Wa.T, Wb.T],
                         axis=1).astype(jnp.bfloat16)
    b5 = jnp.concatenate([bq, bk, bv, ba, bb]).reshape(1, 5 * C)
    xspec = pl.BlockSpec((BB, L, C), lambda j: (0, j, 0))
    out = pl.pallas_call(
        _chunk_kernel,
        grid=grid,
        in_specs=[xspec,
                  pl.BlockSpec((C, 5 * C), lambda j: (0, 0)),
                  pl.BlockSpec((1, 5 * C), lambda j: (0, 0)),
                  pl.BlockSpec((C, C), lambda j: (0, 0)),
                  pl.BlockSpec((1, C), lambda j: (0, 0))],
        out_specs=xspec,
        out_shape=jax.ShapeDtypeStruct((B, N, C), jnp.float32),
        scratch_shapes=[pltpu.VMEM((C, C), jnp.float32) for _ in range(BB)],
        compiler_params=pltpu.CompilerParams(
            dimension_semantics=("arbitrary",)),
    )(x, W5, b5, Wo.T.astype(jnp.bfloat16), bo.reshape(1, C))
    return out
